# baseline (speedup 1.0000x reference)
import functools

import jax
import jax.numpy as jnp
from jax import lax
from jax.experimental import pallas as pl
from jax.experimental.pallas import tpu as pltpu
from jax.experimental.pallas import tpu_sc as plsc

NLAT = 721
NLON = 1440
W = NLON + 1
R = NLAT * W
NQ = 786432
CH = 16

NC = 2
NS = 16
NW = NC * NS
QPT = NQ // NW
B = 512
G = 128
NG = B // G
NCHUNK = QPT // B
CHB = CH * B


def _regrid_sc(table, index, weight):
  mesh = plsc.VectorSubcoreMesh(core_axis_name="c", subcore_axis_name="s")

  @functools.partial(
      pl.kernel,
      out_type=jax.ShapeDtypeStruct((CH * NQ,), jnp.float32),
      mesh=mesh,
      compiler_params=pltpu.CompilerParams(
          needs_layout_passes=False, use_tc_tiling_on_sc=False),
      scratch_types=[
          pltpu.VMEM((B, 4), jnp.int32),
          pltpu.VMEM((B, 4), jnp.float32),
          pltpu.VMEM((4, B), jnp.int32),
          pltpu.VMEM((4, B, CH), jnp.float32),
          pltpu.VMEM((CHB,), jnp.float32),
          pltpu.SemaphoreType.DMA,
          pltpu.SemaphoreType.DMA,
      ],
  )
  def k(table_hbm, idx_hbm, w_hbm, out_hbm, idxc_v, wc_v, idxl_v, rows_v,
        out_v, sem_iw, sem_g):
    wid = lax.axis_index("s") * NC + lax.axis_index("c")
    lanes = lax.iota(jnp.int32, 16)
    c0 = jnp.zeros((16,), jnp.int32)
    c1 = jnp.full((16,), 1, jnp.int32)
    c2 = jnp.full((16,), 2, jnp.int32)
    c3 = jnp.full((16,), 3, jnp.int32)
    corners = (c0, c1, c2, c3)

    def chunk(g, carry):
      base = wid * QPT + g * B
      cp1 = pltpu.async_copy(idx_hbm.at[pl.ds(base, B), :], idxc_v, sem_iw)
      cp2 = pltpu.async_copy(w_hbm.at[pl.ds(base, B), :], wc_v, sem_iw)
      cp1.wait()
      cp2.wait()
      for g16 in range(B // 16):
        qi = g16 * 16 + lanes
        for kk in range(4):
          idxl_v[kk, pl.ds(g16 * 16, 16)] = plsc.load_gather(
              idxc_v, [qi, corners[kk]])
      gps = []
      for kk in range(4):
        for j in range(NG):
          gps.append(pltpu.async_copy(
              table_hbm.at[idxl_v.at[kk, pl.ds(j * G, G)]],
              rows_v.at[kk, pl.ds(j * G, G)], sem_g))
      for gp in gps:
        gp.wait()

      def group(gi, c):
        qb = gi * 16
        qi = qb + lanes
        w0 = plsc.load_gather(wc_v, [qi, c0])
        w1 = plsc.load_gather(wc_v, [qi, c1])
        w2 = plsc.load_gather(wc_v, [qi, c2])
        w3 = plsc.load_gather(wc_v, [qi, c3])
        for ch in range(CH):
          cs = jnp.full((16,), ch, jnp.int32)
          g0 = plsc.load_gather(rows_v.at[0], [qi, cs])
          g1 = plsc.load_gather(rows_v.at[1], [qi, cs])
          g2 = plsc.load_gather(rows_v.at[2], [qi, cs])
          g3 = plsc.load_gather(rows_v.at[3], [qi, cs])
          out_v[pl.ds(ch * B + qb, 16)] = g0 * w0 + g1 * w1 + g2 * w2 + g3 * w3
        return c

      lax.fori_loop(0, B // 16, group, 0)
      pltpu.sync_copy(out_v, out_hbm.at[pl.ds(base * CH, CHB)])
      return carry

    lax.fori_loop(0, NCHUNK, chunk, 0)

  return k(table, index, weight)


def _relayout_tc(out1d):
  NB = NQ // B

  def body(*refs):
    *ins, out = refs
    for c in range(CH):
      out[0, c, :] = ins[c][:]

  in_specs = [
      pl.BlockSpec((B,), lambda j, c=c: (j * CH + c,)) for c in range(CH)
  ]
  return pl.pallas_call(
      body,
      grid=(NB,),
      in_specs=in_specs,
      out_specs=pl.BlockSpec((1, CH, B), lambda j: (0, 0, j)),
      out_shape=jax.ShapeDtypeStruct((1, CH, NQ), jnp.float32),
  )(*([out1d] * CH))


def kernel(x, index, weight):
  xt = jnp.transpose(x[0], (1, 2, 0))
  table = jnp.concatenate([xt, xt[:, :1, :]], axis=1).reshape(R, CH)
  out1d = _regrid_sc(table, index, weight)
  return _relayout_tc(out1d)


# speedup vs baseline: 1.3608x; 1.3608x over previous
import functools

import jax
import jax.numpy as jnp
from jax import lax
from jax.experimental import pallas as pl
from jax.experimental.pallas import tpu as pltpu
from jax.experimental.pallas import tpu_sc as plsc

NLAT = 721
NLON = 1440
W = NLON + 1
R = NLAT * W
NQ = 786432
CH = 16
NQT = NQ // 128

NC = 2
NS = 16
NW = NC * NS
QPT = NQ // NW
B = 512
G = 128
NG = B // G
NCHUNK = QPT // B
CHB = CH * B


def _regrid_sc(table, idx1d, w1d):
  mesh = plsc.VectorSubcoreMesh(core_axis_name="c", subcore_axis_name="s")

  @functools.partial(
      pl.kernel,
      out_type=jax.ShapeDtypeStruct((CH * NQ,), jnp.float32),
      mesh=mesh,
      compiler_params=pltpu.CompilerParams(
          needs_layout_passes=False, use_tc_tiling_on_sc=False),
      scratch_types=[
          pltpu.VMEM((4 * B,), jnp.int32),
          pltpu.VMEM((4 * B,), jnp.float32),
          pltpu.VMEM((4, B), jnp.int32),
          pltpu.VMEM((4, B, CH), jnp.float32),
          pltpu.VMEM((CHB,), jnp.float32),
          pltpu.SemaphoreType.DMA,
          pltpu.SemaphoreType.DMA,
          pltpu.SemaphoreType.DMA,
      ],
  )
  def k(table_hbm, idx_hbm, w_hbm, out_hbm, idxc_v, wc_v, idxl_v, rows_v,
        out_v, sem_iw, sem_g, sem_o):
    wid = lax.axis_index("s") * NC + lax.axis_index("c")
    lanes = lax.iota(jnp.int32, 16)

    def chunk(g, carry):
      base = wid * QPT + g * B
      cp1 = pltpu.async_copy(idx_hbm.at[pl.ds(base * 4, B * 4)], idxc_v,
                             sem_iw)
      cp2 = pltpu.async_copy(w_hbm.at[pl.ds(base * 4, B * 4)], wc_v, sem_iw)
      cp1.wait()
      cp2.wait()
      for g16 in range(B // 16):
        qi4 = (g16 * 64) + lanes * 4
        for kk in range(4):
          idxl_v[kk, pl.ds(g16 * 16, 16)] = plsc.load_gather(
              idxc_v, [qi4 + kk])
      gps = []
      for kk in range(4):
        for j in range(NG):
          gps.append(pltpu.async_copy(
              table_hbm.at[idxl_v.at[kk, pl.ds(j * G, G)]],
              rows_v.at[kk, pl.ds(j * G, G)], sem_g))
      for gp in gps:
        gp.wait()

      def group(gi, c):
        qb = gi * 16
        qi = qb + lanes
        qi4 = qb * 4 + lanes * 4
        w0 = plsc.load_gather(wc_v, [qi4])
        w1 = plsc.load_gather(wc_v, [qi4 + 1])
        w2 = plsc.load_gather(wc_v, [qi4 + 2])
        w3 = plsc.load_gather(wc_v, [qi4 + 3])
        qoff = (qb // 128) * 1024 + (qb % 128)
        for ch in range(CH):
          cs = jnp.full((16,), ch, jnp.int32)
          g0 = plsc.load_gather(rows_v.at[0], [qi, cs])
          g1 = plsc.load_gather(rows_v.at[1], [qi, cs])
          g2 = plsc.load_gather(rows_v.at[2], [qi, cs])
          g3 = plsc.load_gather(rows_v.at[3], [qi, cs])
          off = (ch // 8) * 4096 + (ch % 8) * 128 + qoff
          out_v[pl.ds(off, 16)] = g0 * w0 + g1 * w1 + g2 * w2 + g3 * w3
        return c

      lax.fori_loop(0, B // 16, group, 0)
      J = base // B
      ops = []
      for r in range(2):
        for jql in range(4):
          seg = out_v.at[pl.ds((r * 4 + jql) * 1024, 1024)]
          dst = out_hbm.at[pl.ds((r * NQT + J * 4 + jql) * 1024, 1024)]
          ops.append(pltpu.async_copy(seg, dst, sem_o))
      for op in ops:
        op.wait()
      return carry

    lax.fori_loop(0, NCHUNK, chunk, 0)

  return k(table, idx1d, w1d)


def kernel(x, index, weight):
  xt = jnp.transpose(x[0], (1, 2, 0))
  table = jnp.concatenate([xt, xt[:, :1, :]], axis=1).reshape(R, CH)
  out1d = _regrid_sc(table, index.reshape(-1), weight.reshape(-1))
  out = out1d.reshape(2, NQT, 8, 128).transpose(0, 2, 1, 3)
  return out.reshape(1, CH, NQ)


# speedup vs baseline: 2.9173x; 2.1439x over previous
import functools

import jax
import jax.numpy as jnp
from jax import lax
from jax.experimental import pallas as pl
from jax.experimental.pallas import tpu as pltpu
from jax.experimental.pallas import tpu_sc as plsc

NLAT = 721
NLON = 1440
W = NLON + 1
R = NLAT * W
NQ = 786432
CH = 16
NQT = NQ // 128

NC = 2
NS = 16
NW = NC * NS
QPT = NQ // NW
B = 512
G = 128
NG = B // G
NCHUNK = QPT // B
CHB = CH * B


def _regrid_sc(table, idxT, wT):
  mesh = plsc.VectorSubcoreMesh(core_axis_name="c", subcore_axis_name="s")

  @functools.partial(
      pl.kernel,
      out_type=jax.ShapeDtypeStruct((CH * NQ,), jnp.float32),
      mesh=mesh,
      compiler_params=pltpu.CompilerParams(
          needs_layout_passes=False, use_tc_tiling_on_sc=False),
      scratch_types=[
          pltpu.VMEM((4, B), jnp.int32),
          pltpu.VMEM((4, B), jnp.int32),
          pltpu.VMEM((4, B), jnp.float32),
          pltpu.VMEM((4, B), jnp.float32),
          pltpu.VMEM((4, B, CH), jnp.float32),
          pltpu.VMEM((4, B, CH), jnp.float32),
          pltpu.VMEM((CHB,), jnp.float32),
          pltpu.VMEM((CHB,), jnp.float32),
          pltpu.SemaphoreType.DMA,
          pltpu.SemaphoreType.DMA,
          pltpu.SemaphoreType.DMA,
          pltpu.SemaphoreType.DMA,
          pltpu.SemaphoreType.DMA,
          pltpu.SemaphoreType.DMA,
      ],
  )
  def k(table_hbm, idxT_hbm, wT_hbm, out_hbm, idxl0, idxl1, wl0, wl1,
        rows0, rows1, outv0, outv1, siw0, siw1, sg0, sg1, so0, so1):
    idxl = (idxl0, idxl1)
    wl = (wl0, wl1)
    rows = (rows0, rows1)
    outv = (outv0, outv1)
    siw = (siw0, siw1)
    sg = (sg0, sg1)
    so = (so0, so1)
    wid = lax.axis_index("s") * NC + lax.axis_index("c")
    lanes = lax.iota(jnp.int32, 16)

    def iw_copies(g, sl):
      b = wid * QPT + g * B
      return (pltpu.make_async_copy(idxT_hbm.at[:, pl.ds(b, B)], idxl[sl],
                                    siw[sl]),
              pltpu.make_async_copy(wT_hbm.at[:, pl.ds(b, B)], wl[sl],
                                    siw[sl]))

    def gather_copies(sl):
      return [
          pltpu.make_async_copy(
              table_hbm.at[idxl[sl].at[kk, pl.ds(j * G, G)]],
              rows[sl].at[kk, pl.ds(j * G, G)], sg[sl])
          for kk in range(4) for j in range(NG)
      ]

    def out_copies(g, sl):
      J = wid * NCHUNK + g
      return [
          pltpu.make_async_copy(
              outv[sl].at[pl.ds((r * 4 + jql) * 1024, 1024)],
              out_hbm.at[pl.ds((r * NQT + J * 4 + jql) * 1024, 1024)],
              so[sl])
          for r in range(2) for jql in range(4)
      ]

    def start(cs):
      for c in cs:
        c.start()

    def wait(cs):
      for c in cs:
        c.wait()

    def compute(g, sl):
      rv, wv, ov = rows[sl], wl[sl], outv[sl]

      def group(gi, c):
        qb = gi * 16
        qi = qb + lanes
        w0 = wv[0, pl.ds(qb, 16)]
        w1 = wv[1, pl.ds(qb, 16)]
        w2 = wv[2, pl.ds(qb, 16)]
        w3 = wv[3, pl.ds(qb, 16)]
        qoff = (qb // 128) * 1024 + (qb % 128)
        for ch in range(CH):
          cs = jnp.full((16,), ch, jnp.int32)
          g0 = plsc.load_gather(rv.at[0], [qi, cs])
          g1 = plsc.load_gather(rv.at[1], [qi, cs])
          g2 = plsc.load_gather(rv.at[2], [qi, cs])
          g3 = plsc.load_gather(rv.at[3], [qi, cs])
          off = (ch // 8) * 4096 + (ch % 8) * 128 + qoff
          ov[pl.ds(off, 16)] = g0 * w0 + g1 * w1 + g2 * w2 + g3 * w3
        return c

      lax.fori_loop(0, B // 16, group, 0)

    def body(g, s, has_next=True, has_next2=True, do_owait=True):
      if has_next:
        wait(iw_copies(g + 1, s ^ 1))
        start(gather_copies(s ^ 1))
      wait(gather_copies(s))
      if do_owait:
        wait(out_copies(g, s))
      compute(g, s)
      start(out_copies(g, s))
      if has_next2:
        start(iw_copies(g + 2, s))

    start(iw_copies(0, 0))
    wait(iw_copies(0, 0))
    start(gather_copies(0))
    start(iw_copies(1, 1))
    body(0, 0, do_owait=False)
    body(1, 1, do_owait=False)

    def looped(p, carry):
      body(2 * p, 0)
      body(2 * p + 1, 1)
      return carry

    lax.fori_loop(1, NCHUNK // 2 - 1, looped, 0)

    body(NCHUNK - 2, 0, has_next2=False)
    body(NCHUNK - 1, 1, has_next=False, has_next2=False)
    wait(out_copies(NCHUNK - 2, 0))
    wait(out_copies(NCHUNK - 1, 1))

  return k(table, idxT, wT)


def kernel(x, index, weight):
  xt = jnp.transpose(x[0], (1, 2, 0))
  table = jnp.concatenate([xt, xt[:, :1, :]], axis=1).reshape(R, CH)
  out1d = _regrid_sc(table, index.T, weight.T)
  out = out1d.reshape(2, NQT, 8, 128).transpose(0, 2, 1, 3)
  return out.reshape(1, CH, NQ)


# speedup vs baseline: 3.5254x; 1.2084x over previous
import functools

import jax
import jax.numpy as jnp
from jax import lax
from jax.experimental import pallas as pl
from jax.experimental.pallas import tpu as pltpu
from jax.experimental.pallas import tpu_sc as plsc

NLAT = 721
NLON = 1440
W = NLON + 1
R = NLAT * W
NQ = 786432
CH = 16
NQT = NQ // 128

NC = 2
NS = 16
NW = NC * NS
QPT = NQ // NW
B = 512
G = 128
NG = B // G
NCHUNK = QPT // B
CHB = CH * B


def _regrid_sc(table, idxT, wT):
  mesh = plsc.VectorSubcoreMesh(core_axis_name="c", subcore_axis_name="s")

  @functools.partial(
      pl.kernel,
      out_type=jax.ShapeDtypeStruct((CH * NQ,), jnp.float32),
      name="regrid_gather",
      mesh=mesh,
      compiler_params=pltpu.CompilerParams(
          needs_layout_passes=False, use_tc_tiling_on_sc=False),
      scratch_types=[
          pltpu.VMEM((4, B), jnp.int32),
          pltpu.VMEM((4, B), jnp.int32),
          pltpu.VMEM((4, B), jnp.float32),
          pltpu.VMEM((4, B), jnp.float32),
          pltpu.VMEM((4, B, CH), jnp.float32),
          pltpu.VMEM((4, B, CH), jnp.float32),
          pltpu.VMEM((CHB,), jnp.float32),
          pltpu.VMEM((CHB,), jnp.float32),
          pltpu.SemaphoreType.DMA,
          pltpu.SemaphoreType.DMA,
          pltpu.SemaphoreType.DMA,
          pltpu.SemaphoreType.DMA,
          pltpu.SemaphoreType.DMA,
          pltpu.SemaphoreType.DMA,
      ],
  )
  def k(table_hbm, idxT_hbm, wT_hbm, out_hbm, idxl0, idxl1, wl0, wl1,
        rows0, rows1, outv0, outv1, siw0, siw1, sg0, sg1, so0, so1):
    idxl = (idxl0, idxl1)
    wl = (wl0, wl1)
    rows = (rows0, rows1)
    outv = (outv0, outv1)
    siw = (siw0, siw1)
    sg = (sg0, sg1)
    so = (so0, so1)
    wid = lax.axis_index("s") * NC + lax.axis_index("c")
    lanes = lax.iota(jnp.int32, 16)

    def iw_copies(g, sl):
      b = wid * QPT + g * B
      return (pltpu.make_async_copy(idxT_hbm.at[:, pl.ds(b, B)], idxl[sl],
                                    siw[sl]),
              pltpu.make_async_copy(wT_hbm.at[:, pl.ds(b, B)], wl[sl],
                                    siw[sl]))

    def gather_copies(sl):
      return [
          pltpu.make_async_copy(
              table_hbm.at[idxl[sl].at[kk, pl.ds(j * G, G)]],
              rows[sl].at[kk, pl.ds(j * G, G)], sg[sl])
          for kk in range(4) for j in range(NG)
      ]

    def out_copies(g, sl):
      J = wid * NCHUNK + g
      return [
          pltpu.make_async_copy(
              outv[sl].at[pl.ds((r * 4 + jql) * 1024, 1024)],
              out_hbm.at[pl.ds((r * NQT + J * 4 + jql) * 1024, 1024)],
              so[sl])
          for r in range(2) for jql in range(4)
      ]

    def start(cs):
      for c in cs:
        c.start()

    def wait(cs):
      for c in cs:
        c.wait()

    def compute(g, sl):
      rv, wv, ov = rows[sl], wl[sl], outv[sl]

      def group(gi, c):
        qb = gi * 16
        qi = qb + lanes
        w0 = wv[0, pl.ds(qb, 16)]
        w1 = wv[1, pl.ds(qb, 16)]
        w2 = wv[2, pl.ds(qb, 16)]
        w3 = wv[3, pl.ds(qb, 16)]
        qoff = (qb // 128) * 1024 + (qb % 128)
        for ch in range(0, CH, 2):
          ca = jnp.full((16,), ch, jnp.int32)
          cb = jnp.full((16,), ch + 1, jnp.int32)
          a0 = plsc.load_gather(rv.at[0], [qi, ca])
          b0 = plsc.load_gather(rv.at[0], [qi, cb])
          a1 = plsc.load_gather(rv.at[1], [qi, ca])
          b1 = plsc.load_gather(rv.at[1], [qi, cb])
          a2 = plsc.load_gather(rv.at[2], [qi, ca])
          b2 = plsc.load_gather(rv.at[2], [qi, cb])
          a3 = plsc.load_gather(rv.at[3], [qi, ca])
          b3 = plsc.load_gather(rv.at[3], [qi, cb])
          offa = (ch // 8) * 4096 + (ch % 8) * 128 + qoff
          offb = ((ch + 1) // 8) * 4096 + ((ch + 1) % 8) * 128 + qoff
          ov[pl.ds(offa, 16)] = a0 * w0 + a1 * w1 + a2 * w2 + a3 * w3
          ov[pl.ds(offb, 16)] = b0 * w0 + b1 * w1 + b2 * w2 + b3 * w3
        return c

      lax.fori_loop(0, B // 16, group, 0)

    def body(g, s, has_next=True, has_next2=True, do_owait=True):
      if has_next:
        wait(iw_copies(g + 1, s ^ 1))
        start(gather_copies(s ^ 1))
      wait(gather_copies(s))
      if do_owait:
        wait(out_copies(g, s))
      compute(g, s)
      start(out_copies(g, s))
      if has_next2:
        start(iw_copies(g + 2, s))

    start(iw_copies(0, 0))
    wait(iw_copies(0, 0))
    start(gather_copies(0))
    start(iw_copies(1, 1))
    body(0, 0, do_owait=False)
    body(1, 1, do_owait=False)

    def looped(p, carry):
      body(2 * p, 0)
      body(2 * p + 1, 1)
      return carry

    lax.fori_loop(1, NCHUNK // 2 - 1, looped, 0)

    body(NCHUNK - 2, 0, has_next2=False)
    body(NCHUNK - 1, 1, has_next=False, has_next2=False)
    wait(out_copies(NCHUNK - 2, 0))
    wait(out_copies(NCHUNK - 1, 1))

  return k(table, idxT, wT)


def kernel(x, index, weight):
  xt = jnp.transpose(x[0], (1, 2, 0))
  table = jnp.concatenate([xt, xt[:, :1, :]], axis=1).reshape(R, CH)
  out1d = _regrid_sc(table, index.T, weight.T)
  out = out1d.reshape(2, NQT, 8, 128).transpose(0, 2, 1, 3)
  return out.reshape(1, CH, NQ)
